# Initial kernel scaffold; baseline (speedup 1.0000x reference)
import jax
import jax.numpy as jnp
from jax.experimental import pallas as pl


def kernel(x, emb_usr):
    raise NotImplementedError("write your pallas kernel here")


# speedup vs baseline: 10.3235x; 10.3235x over previous
import functools

import jax
import jax.numpy as jnp
from jax import lax
from jax.experimental import pallas as pl
from jax.experimental.pallas import tpu as pltpu
from jax.experimental.pallas import tpu_sc as plsc

_EMB = 32
_L = 16
_NC = 2
_NS = 16
_NW = _NC * _NS
_CH = 128


def _make_gather(bh: int):
    b_per_w = bh // _NW
    n_chunks = b_per_w // _CH
    k_vec = _CH // _L

    mesh = plsc.VectorSubcoreMesh(core_axis_name="c", subcore_axis_name="s")

    @functools.partial(
        pl.kernel,
        mesh=mesh,
        compiler_params=pltpu.CompilerParams(use_tc_tiling_on_sc=False),
        out_type=jax.ShapeDtypeStruct((bh, _EMB), jnp.float32),
        scratch_types=[
            pltpu.VMEM((n_chunks, _CH), jnp.int32),
            pltpu.VMEM((_CH, _EMB), jnp.float32),
            pltpu.SemaphoreType.DMA,
        ],
    )
    def gather_kernel(x_hbm, table_hbm, out_hbm, idx_v, rows_v, sem):
        wid = lax.axis_index("s") * _NC + lax.axis_index("c")
        base = wid * b_per_w
        pltpu.sync_copy(x_hbm.at[wid], idx_v)

        def chunk_body(c, carry):
            def add1(k, carry2):
                sl = pl.ds(k * _L, _L)
                idx_v[c, sl] = idx_v[c, sl] + 1
                return carry2

            lax.fori_loop(0, k_vec, add1, 0, unroll=True)
            pltpu.async_copy(table_hbm.at[idx_v.at[c]], rows_v, sem).wait()
            pltpu.sync_copy(rows_v, out_hbm.at[pl.ds(base + c * _CH, _CH)])
            return carry

        lax.fori_loop(0, n_chunks, chunk_body, 0)

    return gather_kernel


def kernel(x, emb_usr):
    batch, hist = x.shape
    bh = batch * hist
    x3 = x.reshape(_NW, bh // (_NW * _CH), _CH)
    out = _make_gather(bh)(x3, emb_usr)
    return out.reshape(batch, hist, _EMB)


# speedup vs baseline: 10.7981x; 1.0460x over previous
import functools

import jax
import jax.numpy as jnp
from jax import lax
from jax.experimental import pallas as pl
from jax.experimental.pallas import tpu as pltpu
from jax.experimental.pallas import tpu_sc as plsc

_EMB = 32
_L = 16
_NC = 2
_NS = 16
_NW = _NC * _NS
_CH = 128


_GROUP = 5
_ROWS_G = _GROUP * _CH
_NBUF = 2


def _make_gather(bh: int):
    b_per_w = bh // _NW
    n_chunks = b_per_w // _CH
    n_groups = b_per_w // _ROWS_G
    n_outer = n_groups // _NBUF
    k_vec = _CH // _L

    mesh = plsc.VectorSubcoreMesh(core_axis_name="c", subcore_axis_name="s")

    @functools.partial(
        pl.kernel,
        mesh=mesh,
        compiler_params=pltpu.CompilerParams(use_tc_tiling_on_sc=False),
        out_type=jax.ShapeDtypeStruct((bh, _EMB), jnp.float32),
        scratch_types=[
            pltpu.VMEM((n_chunks, _CH), jnp.int32),
            pltpu.VMEM((_NBUF, _ROWS_G, _EMB), jnp.float32),
            pltpu.SemaphoreType.DMA,
            pltpu.SemaphoreType.DMA,
            pltpu.SemaphoreType.DMA,
            pltpu.SemaphoreType.DMA,
        ],
    )
    def gather_kernel(x_hbm, table_hbm, out_hbm, idx_v, rows_v,
                      gsem0, gsem1, wsem0, wsem1):
        gsems = (gsem0, gsem1)
        wsems = (wsem0, wsem1)
        wid = lax.axis_index("s") * _NC + lax.axis_index("c")
        base = wid * b_per_w
        pltpu.sync_copy(x_hbm.at[wid], idx_v)

        def fire(g, buf):
            for j in range(_GROUP):
                c = g * _GROUP + j
                for k in range(k_vec):
                    sl = pl.ds(k * _L, _L)
                    idx_v[c, sl] = idx_v[c, sl] + 1
                pltpu.async_copy(
                    table_hbm.at[idx_v.at[c]],
                    rows_v.at[buf, pl.ds(j * _CH, _CH)],
                    gsems[buf])

        def drain_gathers(buf):
            pltpu.make_async_copy(
                table_hbm.at[pl.ds(0, _ROWS_G)], rows_v.at[buf],
                gsems[buf]).wait()

        def write(g, buf):
            pltpu.async_copy(
                rows_v.at[buf],
                out_hbm.at[pl.ds(base + g * _ROWS_G, _ROWS_G)],
                wsems[buf])

        def drain_write(buf):
            pltpu.make_async_copy(
                rows_v.at[buf], out_hbm.at[pl.ds(base, _ROWS_G)],
                wsems[buf]).wait()

        fire(0, 0)
        fire(1, 1)

        def outer(o, carry):
            g0 = o * _NBUF
            @pl.when(o > 0)
            def _():
                drain_write(1)
                fire(g0 + 1, 1)

            drain_gathers(0)
            write(g0, 0)
            drain_write(0)

            @pl.when(o < n_outer - 1)
            def _():
                fire(g0 + 2, 0)

            drain_gathers(1)
            write(g0 + 1, 1)
            return carry

        lax.fori_loop(0, n_outer, outer, 0)
        drain_write(1)

    return gather_kernel


def kernel(x, emb_usr):
    batch, hist = x.shape
    bh = batch * hist
    x3 = x.reshape(_NW, bh // (_NW * _CH), _CH)
    out = _make_gather(bh)(x3, emb_usr)
    return out.reshape(batch, hist, _EMB)


# speedup vs baseline: 12.3092x; 1.1399x over previous
import functools

import jax
import jax.numpy as jnp
from jax import lax
from jax.experimental import pallas as pl
from jax.experimental.pallas import tpu as pltpu
from jax.experimental.pallas import tpu_sc as plsc

_EMB = 32
_L = 16
_NC = 2
_NS = 16
_NW = _NC * _NS
_CH = 128


def _make_gather(batch: int, hist: int):
    assert batch % (_NW * _CH) == 0 or batch == _NW * _CH
    k_vec = _CH // _L
    n_pairs = hist // 2

    mesh = plsc.VectorSubcoreMesh(core_axis_name="c", subcore_axis_name="s")

    @functools.partial(
        pl.kernel,
        mesh=mesh,
        compiler_params=pltpu.CompilerParams(
            use_tc_tiling_on_sc=False, needs_layout_passes=False),
        out_type=jax.ShapeDtypeStruct((hist, _EMB, batch), jnp.float32),
        scratch_types=[
            pltpu.VMEM((hist, _CH), jnp.int32),
            pltpu.VMEM((2, _CH, _EMB), jnp.float32),
            pltpu.VMEM((2, _EMB, _CH), jnp.float32),
            pltpu.SemaphoreType.DMA,
            pltpu.SemaphoreType.DMA,
            pltpu.SemaphoreType.DMA,
            pltpu.SemaphoreType.DMA,
        ],
    )
    def gather_kernel(xt_hbm, table_hbm, out_hbm, idx_v, rows_v, tbuf_v,
                      gsem0, gsem1, wsem0, wsem1):
        gsems = (gsem0, gsem1)
        wsems = (wsem0, wsem1)
        wid = lax.axis_index("s") * _NC + lax.axis_index("c")
        base_b = wid * _CH
        pltpu.sync_copy(xt_hbm.at[:, pl.ds(base_b, _CH)], idx_v)

        def prep(h):
            for k in range(k_vec):
                sl = pl.ds(k * _L, _L)
                idx_v[h, sl] = idx_v[h, sl] + 1

        def fire(h, buf):
            pltpu.async_copy(
                table_hbm.at[idx_v.at[h]], rows_v.at[buf], gsems[buf])

        def wait_gather(buf):
            pltpu.make_async_copy(
                table_hbm.at[pl.ds(0, _CH)], rows_v.at[buf],
                gsems[buf]).wait()

        def transpose(buf):
            iota = lax.iota(jnp.int32, _L)

            def tbody(c, carry):
                cvec = jnp.full((_L,), c, jnp.int32)
                for k in range(k_vec):
                    bidx = iota + (k * _L)
                    val = plsc.load_gather(rows_v.at[buf], [bidx, cvec])
                    tbuf_v[buf, c, pl.ds(k * _L, _L)] = val
                return carry

            lax.fori_loop(0, _EMB, tbody, 0)

        def write(h, buf):
            pltpu.async_copy(
                tbuf_v.at[buf],
                out_hbm.at[h, :, pl.ds(base_b, _CH)],
                wsems[buf])

        def wait_write(buf):
            pltpu.make_async_copy(
                tbuf_v.at[buf], out_hbm.at[0, :, pl.ds(base_b, _CH)],
                wsems[buf]).wait()

        prep(0)
        fire(0, 0)

        def outer(o, carry):
            h0 = 2 * o
            prep(h0 + 1)
            fire(h0 + 1, 1)
            wait_gather(0)

            @pl.when(o > 0)
            def _():
                wait_write(0)

            transpose(0)
            write(h0, 0)

            @pl.when(o < n_pairs - 1)
            def _():
                prep(h0 + 2)
                fire(h0 + 2, 0)

            wait_gather(1)

            @pl.when(o > 0)
            def _():
                wait_write(1)

            transpose(1)
            write(h0 + 1, 1)
            return carry

        lax.fori_loop(0, n_pairs, outer, 0)
        wait_write(0)
        wait_write(1)

    return gather_kernel


def kernel(x, emb_usr):
    batch, hist = x.shape
    xt = x.T
    out_t = _make_gather(batch, hist)(xt, emb_usr)
    return out_t.transpose(2, 0, 1)


# speedup vs baseline: 13.3590x; 1.0853x over previous
import functools

import jax
import jax.numpy as jnp
from jax import lax
from jax.experimental import pallas as pl
from jax.experimental.pallas import tpu as pltpu
from jax.experimental.pallas import tpu_sc as plsc

_EMB = 32
_L = 16
_NC = 2
_NS = 16
_NW = _NC * _NS
_CH = 128


def _make_gather(batch: int, hist: int):
    k_vec = _CH // _L
    n_pairs = hist // 2

    mesh = plsc.VectorSubcoreMesh(core_axis_name="c", subcore_axis_name="s")

    @functools.partial(
        pl.kernel,
        mesh=mesh,
        compiler_params=pltpu.CompilerParams(
            use_tc_tiling_on_sc=False, needs_layout_passes=False),
        out_type=jax.ShapeDtypeStruct((hist, batch, _EMB), jnp.float32),
        scratch_types=[
            pltpu.VMEM((hist, _CH), jnp.int32),
            pltpu.VMEM((2, _CH, _EMB), jnp.float32),
            pltpu.SemaphoreType.DMA,
            pltpu.SemaphoreType.DMA,
            pltpu.SemaphoreType.DMA,
            pltpu.SemaphoreType.DMA,
        ],
    )
    def gather_kernel(xt_hbm, table_hbm, out_hbm, idx_v, rows_v,
                      gsem0, gsem1, wsem0, wsem1):
        gsems = (gsem0, gsem1)
        wsems = (wsem0, wsem1)
        wid = lax.axis_index("s") * _NC + lax.axis_index("c")
        base_b = wid * _CH
        pltpu.sync_copy(xt_hbm.at[:, pl.ds(base_b, _CH)], idx_v)

        def prep(h):
            for k in range(k_vec):
                sl = pl.ds(k * _L, _L)
                idx_v[h, sl] = idx_v[h, sl] + 1

        def fire(h, buf):
            pltpu.async_copy(
                table_hbm.at[idx_v.at[h]], rows_v.at[buf], gsems[buf])

        def wait_gather(buf):
            pltpu.make_async_copy(
                table_hbm.at[pl.ds(0, _CH)], rows_v.at[buf],
                gsems[buf]).wait()

        def write(h, buf):
            pltpu.async_copy(
                rows_v.at[buf],
                out_hbm.at[h, pl.ds(base_b, _CH)],
                wsems[buf])

        def wait_write(buf):
            pltpu.make_async_copy(
                rows_v.at[buf], out_hbm.at[0, pl.ds(base_b, _CH)],
                wsems[buf]).wait()

        prep(0)
        fire(0, 0)
        prep(1)
        fire(1, 1)

        def outer(o, carry):
            h0 = 2 * o
            wait_gather(0)
            write(h0, 0)

            @pl.when(o < n_pairs - 1)
            def _():
                prep(h0 + 2)
                wait_write(0)
                fire(h0 + 2, 0)

            wait_gather(1)
            write(h0 + 1, 1)

            @pl.when(o < n_pairs - 1)
            def _():
                prep(h0 + 3)
                wait_write(1)
                fire(h0 + 3, 1)

            return carry

        lax.fori_loop(0, n_pairs, outer, 0)
        wait_write(0)
        wait_write(1)

    return gather_kernel


def kernel(x, emb_usr):
    batch, hist = x.shape
    xt = x.T
    out_t = _make_gather(batch, hist)(xt, emb_usr)
    return out_t.transpose(1, 0, 2)
